# baseline (speedup 1.0000x reference)
import functools

import jax
import jax.numpy as jnp
from jax import lax
from jax.experimental import pallas as pl
from jax.experimental.pallas import tpu as pltpu
from jax.experimental.pallas import tpu_sc as plsc

N = 100000
E = 1600000
F_IN = 33
H = 64
G = 512

NC, NS = 2, 16
NW = NC * NS
CW = 16
NCHUNK = H // CW
EP = 1638400
EG = EP // 128
GPS = EG // NS
K = 4
NBLK = GPS // K
NPAIR = NBLK // 2
NPAD = 100096
RPS = NPAD // NS
ZCH = 184

NGF = N // 128
GPW = 24
NGR = NGF - NW * GPW
TAIL = N - NGF * 128

_sc_mesh = plsc.VectorSubcoreMesh(core_axis_name="c", subcore_axis_name="s")


def _agg_body(z0, z1, z2, z3, src_g, dst_g,
              o0, o1, o2, o3,
              srcv0, dstv0, srcv1, dstv1, rows0, rows1, zv, acc,
              gsem, ssem):
    cid = lax.axis_index("c")
    sid = lax.axis_index("s")
    zs = (z0, z1, z2, z3)
    outs = (o0, o1, o2, o3)

    def _zfill(i, _):
        zv[i, :] = jnp.zeros((CW,), jnp.float32)
        return 0
    lax.fori_loop(0, ZCH, _zfill, 0)

    for jj in range(NCHUNK):
        @pl.when(cid == (jj % NC))
        def _chunk(jj=jj):
            zt = zs[jj]
            ot = outs[jj]
            for t in range(RPS // ZCH):
                pltpu.sync_copy(zv, acc.at[pl.ds(sid * RPS + t * ZCH, ZCH)])
            plsc.subcore_barrier()

            def load_and_fire(b, sv, dv, rv):
                g0 = sid * GPS + b * K
                pltpu.sync_copy(src_g.at[pl.ds(g0, K)], sv)
                pltpu.sync_copy(dst_g.at[pl.ds(g0, K)], dv)
                for k in range(K):
                    pltpu.async_copy(zt.at[sv.at[k]],
                                     rv.at[pl.ds(k * 128, 128)], gsem)

            def drain_gathers(sv, rv):
                for k in range(K):
                    pltpu.make_async_copy(
                        zt.at[sv.at[k]],
                        rv.at[pl.ds(k * 128, 128)], gsem).wait()

            def fire_scatters(dv, rv):
                for k in range(K):
                    pltpu.async_copy(rv.at[pl.ds(k * 128, 128)],
                                     acc.at[dv.at[k]], ssem, add=True)

            def drain_scatters(dv, rv):
                for k in range(K):
                    pltpu.make_async_copy(rv.at[pl.ds(k * 128, 128)],
                                          acc.at[dv.at[k]], ssem).wait()

            load_and_fire(0, srcv0, dstv0, rows0)

            def pip(i, _):
                drain_gathers(srcv0, rows0)
                fire_scatters(dstv0, rows0)

                @pl.when(i > 0)
                def _d1():
                    drain_scatters(dstv1, rows1)
                load_and_fire(2 * i + 1, srcv1, dstv1, rows1)
                drain_gathers(srcv1, rows1)
                fire_scatters(dstv1, rows1)
                drain_scatters(dstv0, rows0)

                @pl.when(i < NPAIR - 1)
                def _n0():
                    load_and_fire(2 * i + 2, srcv0, dstv0, rows0)
                return 0
            lax.fori_loop(0, NPAIR, pip, 0)
            drain_scatters(dstv1, rows1)
            plsc.subcore_barrier()

            for t in range(RPS // ZCH):
                r0 = sid * RPS + t * ZCH
                pltpu.sync_copy(acc.at[pl.ds(r0, ZCH)],
                                rows0.at[pl.ds(0, ZCH)])
                pltpu.sync_copy(rows0.at[pl.ds(0, ZCH)],
                                ot.at[pl.ds(r0, ZCH)])
            plsc.subcore_barrier()


_agg = pl.kernel(
    _agg_body,
    out_type=[jax.ShapeDtypeStruct((NPAD, CW), jnp.float32)] * NCHUNK,
    mesh=_sc_mesh,
    scratch_types=[
        pltpu.VMEM((K, 128), jnp.int32),
        pltpu.VMEM((K, 128), jnp.int32),
        pltpu.VMEM((K, 128), jnp.int32),
        pltpu.VMEM((K, 128), jnp.int32),
        pltpu.VMEM((K * 128, CW), jnp.float32),
        pltpu.VMEM((K * 128, CW), jnp.float32),
        pltpu.VMEM((ZCH, CW), jnp.float32),
        pltpu.VMEM_SHARED((NPAD, CW), jnp.float32),
        pltpu.SemaphoreType.DMA,
        pltpu.SemaphoreType.DMA,
    ],
    compiler_params=pltpu.CompilerParams(use_tc_tiling_on_sc=False),
    name="gin_agg",
)


def _pool_body(h, bat_g, bat_t, out, hbuf, bidx, tidx, zp, pacc, psem):
    del psem
    cid = lax.axis_index("c")
    sid = lax.axis_index("s")
    w = sid * NC + cid

    for i in range(16):
        for c in range(4):
            zp[i, pl.ds(c * 16, 16)] = jnp.zeros((16,), jnp.float32)
    pltpu.sync_copy(zp, pacc.at[pl.ds(sid * 32, 16)])
    pltpu.sync_copy(zp, pacc.at[pl.ds(sid * 32 + 16, 16)])
    plsc.subcore_barrier()

    pltpu.sync_copy(bat_g.at[pl.ds(w * GPW, GPW)], bidx)

    def gb(i, _):
        g = w * GPW + i
        pltpu.sync_copy(h.at[pl.ds(g * 128, 128)], hbuf)
        pltpu.sync_copy(hbuf, pacc.at[bidx.at[i]], add=True)
        return 0
    lax.fori_loop(0, GPW, gb, 0)

    @pl.when(w == 0)
    def _rem():
        pltpu.sync_copy(bat_g.at[pl.ds(NW * GPW, NGR)],
                        bidx.at[pl.ds(0, NGR)])

        def rb(i, _):
            g = NW * GPW + i
            pltpu.sync_copy(h.at[pl.ds(g * 128, 128)], hbuf)
            pltpu.sync_copy(hbuf, pacc.at[bidx.at[i]], add=True)
            return 0
        lax.fori_loop(0, NGR, rb, 0)

    @pl.when(w == NW - 1)
    def _tail():
        pltpu.sync_copy(h.at[pl.ds(NGF * 128, TAIL)], hbuf.at[pl.ds(0, TAIL)])
        pltpu.sync_copy(bat_t, tidx)
        pltpu.sync_copy(hbuf.at[pl.ds(0, TAIL)], pacc.at[tidx.at[0]],
                        add=True)
    plsc.subcore_barrier()

    pltpu.sync_copy(pacc.at[pl.ds(sid * 32, 32)], hbuf.at[pl.ds(0, 32)])
    for cc in range(NC):
        @pl.when(cid == cc)
        def _flush(cc=cc):
            pltpu.sync_copy(hbuf.at[pl.ds(0, 32)],
                            out.at[cc].at[pl.ds(sid * 32, 32)])


_pool = pl.kernel(
    _pool_body,
    out_type=jax.ShapeDtypeStruct((NC, G, H), jnp.float32),
    mesh=_sc_mesh,
    scratch_types=[
        pltpu.VMEM((128, H), jnp.float32),
        pltpu.VMEM((GPW, 128), jnp.int32),
        pltpu.VMEM((1, TAIL), jnp.int32),
        pltpu.VMEM((16, H), jnp.float32),
        pltpu.VMEM_SHARED((G, H), jnp.float32),
        pltpu.SemaphoreType.DMA,
    ],
    compiler_params=pltpu.CompilerParams(use_tc_tiling_on_sc=False),
    name="gin_pool",
)


BN = 800


def _lin1_body(x_ref, w_ref, o0, o1, o2, o3):
    z = jnp.dot(x_ref[...], w_ref[...], preferred_element_type=jnp.float32)
    for c, o in enumerate((o0, o1, o2, o3)):
        o[...] = z[:, c * CW:(c + 1) * CW]


_lin1 = pl.pallas_call(
    _lin1_body,
    grid=(N // BN,),
    in_specs=[pl.BlockSpec((BN, F_IN), lambda i: (i, 0)),
              pl.BlockSpec((F_IN, H), lambda i: (0, 0))],
    out_specs=[pl.BlockSpec((BN, CW), lambda i: (i, 0))] * NCHUNK,
    out_shape=[jax.ShapeDtypeStruct((N, CW), jnp.float32)] * NCHUNK,
)


def _ep_core(z, a, b1r, scr, ber, w2r, b2r):
    pre = (z + a + b1r[...]) * scr[...] + ber[...]
    t = jnp.maximum(pre, 0.0)
    u = jnp.dot(t, w2r[...], preferred_element_type=jnp.float32) + b2r[...]
    return jnp.maximum(u, 0.0)


def _ep_mid_body(z0, z1, z2, z3, a0, a1, a2, a3,
                 b1r, scr, ber, w2r, b2r, w1nr,
                 hr, n0, n1, n2, n3):
    z = jnp.concatenate([z0[...], z1[...], z2[...], z3[...]], axis=1)
    a = jnp.concatenate([a0[...], a1[...], a2[...], a3[...]], axis=1)
    u = _ep_core(z, a, b1r, scr, ber, w2r, b2r)
    hr[...] = u
    zn = jnp.dot(u, w1nr[...], preferred_element_type=jnp.float32)
    for c, o in enumerate((n0, n1, n2, n3)):
        o[...] = zn[:, c * CW:(c + 1) * CW]


def _ep_last_body(z0, z1, z2, z3, a0, a1, a2, a3,
                  b1r, scr, ber, w2r, b2r, hr):
    z = jnp.concatenate([z0[...], z1[...], z2[...], z3[...]], axis=1)
    a = jnp.concatenate([a0[...], a1[...], a2[...], a3[...]], axis=1)
    hr[...] = _ep_core(z, a, b1r, scr, ber, w2r, b2r)


_chunk_spec = pl.BlockSpec((BN, CW), lambda i: (i, 0))
_vec_spec = pl.BlockSpec((1, H), lambda i: (0, 0))
_mat_spec = pl.BlockSpec((H, H), lambda i: (0, 0))

_ep_mid = pl.pallas_call(
    _ep_mid_body,
    grid=(N // BN,),
    in_specs=[_chunk_spec] * 8 + [_vec_spec] * 3 + [_mat_spec, _vec_spec,
                                                    _mat_spec],
    out_specs=[pl.BlockSpec((BN, H), lambda i: (i, 0))] + [_chunk_spec] * 4,
    out_shape=[jax.ShapeDtypeStruct((N, H), jnp.float32)]
    + [jax.ShapeDtypeStruct((N, CW), jnp.float32)] * NCHUNK,
)

_ep_last = pl.pallas_call(
    _ep_last_body,
    grid=(N // BN,),
    in_specs=[_chunk_spec] * 8 + [_vec_spec] * 3 + [_mat_spec, _vec_spec],
    out_specs=pl.BlockSpec((BN, H), lambda i: (i, 0)),
    out_shape=jax.ShapeDtypeStruct((N, H), jnp.float32),
)


def _fin_body(p1r, p2r, p3r, w1r, b1r, w2r, b2r, w3r, b3r, outr):
    ps = [r[...][0] + r[...][1] for r in (p1r, p2r, p3r)]
    hcat = jnp.concatenate(ps, axis=1)
    t = jnp.dot(hcat, w1r[...], preferred_element_type=jnp.float32) + b1r[...]
    t = jnp.maximum(t, 0.0)
    t = jnp.dot(t, w2r[...], preferred_element_type=jnp.float32) + b2r[...]
    t = jnp.maximum(t, 0.0)
    t = jnp.dot(t, w3r[...], preferred_element_type=jnp.float32) + b3r[...]
    outr[...] = 1.0 / (1.0 + jnp.exp(-t))


def _full(shape):
    return pl.BlockSpec(shape, lambda: tuple(0 for _ in shape))


_fin = pl.pallas_call(
    _fin_body,
    in_specs=[_full((NC, G, H))] * 3
    + [_full((3 * H, 50)), _full((1, 50)), _full((50, 20)), _full((1, 20)),
       _full((20, 1)), _full((1, 1))],
    out_specs=_full((G, 1)),
    out_shape=jax.ShapeDtypeStruct((G, 1), jnp.float32),
)


def kernel(x, edge_index, batch,
           W1_1, b1_1, gamma_1, beta_1, W2_1, b2_1,
           W1_2, b1_2, gamma_2, beta_2, W2_2, b2_2,
           W1_3, b1_3, gamma_3, beta_3, W2_3, b2_3,
           Wl1, bl1, Wl2, bl2, Wl3, bl3):
    src = edge_index[0]
    dst = edge_index[1]
    pad = EP - E
    src_g = jnp.concatenate([src, jnp.zeros((pad,), jnp.int32)]).reshape(
        EG, 128)
    dst_g = jnp.concatenate([dst, jnp.full((pad,), N, jnp.int32)]).reshape(
        EG, 128)
    bat_g = batch[:NGF * 128].reshape(NGF, 128)
    bat_t = batch[NGF * 128:].reshape(1, TAIL)

    inv = 1.0 / jnp.sqrt(jnp.float32(1.0 + 1e-5))
    r = lambda v: v.reshape(1, -1)

    z1 = _lin1(x, W1_1)
    a1 = _agg(*z1, src_g, dst_g)
    h1, *z2 = _ep_mid(*z1, *a1, r(b1_1), r(gamma_1 * inv), r(beta_1),
                      W2_1, r(b2_1), W1_2)
    a2 = _agg(*z2, src_g, dst_g)
    h2, *z3 = _ep_mid(*z2, *a2, r(b1_2), r(gamma_2 * inv), r(beta_2),
                      W2_2, r(b2_2), W1_3)
    a3 = _agg(*z3, src_g, dst_g)
    h3 = _ep_last(*z3, *a3, r(b1_3), r(gamma_3 * inv), r(beta_3),
                  W2_3, r(b2_3))
    p1 = _pool(h1, bat_g, bat_t)
    p2 = _pool(h2, bat_g, bat_t)
    p3 = _pool(h3, bat_g, bat_t)
    return _fin(p1, p2, p3, Wl1, r(bl1), Wl2, r(bl2), Wl3, r(bl3))


# speedup vs baseline: 1.1780x; 1.1780x over previous
import functools

import jax
import jax.numpy as jnp
from jax import lax
from jax.experimental import pallas as pl
from jax.experimental.pallas import tpu as pltpu
from jax.experimental.pallas import tpu_sc as plsc

N = 100000
E = 1600000
F_IN = 33
H = 64
G = 512

NC, NS = 2, 16
NW = NC * NS
CW = 16
NCHUNK = H // CW
EP = 1638400
EG = EP // 128
GPS = EG // NS
K = 4
NBLK = GPS // K
NPAIR = NBLK // 2
NPAD = 100096
RPS = NPAD // NS
ZCH = 184

NGF = N // 128
GPW = 24
NGR = NGF - NW * GPW
TAIL = N - NGF * 128

_sc_mesh = plsc.VectorSubcoreMesh(core_axis_name="c", subcore_axis_name="s")


def _agg_body(z0, z1, z2, z3, src_g, dst_g,
              o0, o1, o2, o3,
              srcv0, dstv0, srcv1, dstv1, rows0, rows1, zv, acc,
              gsem, ssem):
    cid = lax.axis_index("c")
    sid = lax.axis_index("s")
    zs = (z0, z1, z2, z3)
    outs = (o0, o1, o2, o3)

    def _zfill(i, _):
        zv[i, :] = jnp.zeros((CW,), jnp.float32)
        return 0
    lax.fori_loop(0, ZCH, _zfill, 0)

    for jj in range(NCHUNK):
        @pl.when(cid == (jj % NC))
        def _chunk(jj=jj):
            zt = zs[jj]
            ot = outs[jj]
            for t in range(RPS // ZCH):
                pltpu.sync_copy(zv, acc.at[pl.ds(sid * RPS + t * ZCH, ZCH)])
            plsc.subcore_barrier()

            def load_and_fire(b, sv, dv, rv):
                g0 = sid * GPS + b * K
                pltpu.sync_copy(src_g.at[pl.ds(g0, K)], sv)
                pltpu.sync_copy(dst_g.at[pl.ds(g0, K)], dv)
                for k in range(K):
                    pltpu.async_copy(zt.at[sv.at[k]],
                                     rv.at[pl.ds(k * 128, 128)], gsem)

            def drain_gathers(sv, rv):
                for k in range(K):
                    pltpu.make_async_copy(
                        zt.at[sv.at[k]],
                        rv.at[pl.ds(k * 128, 128)], gsem).wait()

            def fire_scatters(dv, rv):
                for k in range(K):
                    pltpu.async_copy(rv.at[pl.ds(k * 128, 128)],
                                     acc.at[dv.at[k]], ssem, add=True)

            def drain_scatters(dv, rv):
                for k in range(K):
                    pltpu.make_async_copy(rv.at[pl.ds(k * 128, 128)],
                                          acc.at[dv.at[k]], ssem).wait()

            load_and_fire(0, srcv0, dstv0, rows0)

            def pip(i, _):
                @pl.when(i > 0)
                def _d1():
                    drain_scatters(dstv1, rows1)
                load_and_fire(2 * i + 1, srcv1, dstv1, rows1)
                drain_gathers(srcv0, rows0)
                fire_scatters(dstv0, rows0)

                @pl.when(i < NPAIR - 1)
                def _n0():
                    drain_scatters(dstv0, rows0)
                    load_and_fire(2 * i + 2, srcv0, dstv0, rows0)
                drain_gathers(srcv1, rows1)
                fire_scatters(dstv1, rows1)
                return 0
            lax.fori_loop(0, NPAIR, pip, 0)
            drain_scatters(dstv0, rows0)
            drain_scatters(dstv1, rows1)
            plsc.subcore_barrier()

            for t in range(RPS // ZCH):
                r0 = sid * RPS + t * ZCH
                pltpu.sync_copy(acc.at[pl.ds(r0, ZCH)],
                                rows0.at[pl.ds(0, ZCH)])
                pltpu.sync_copy(rows0.at[pl.ds(0, ZCH)],
                                ot.at[pl.ds(r0, ZCH)])
            plsc.subcore_barrier()


_agg = pl.kernel(
    _agg_body,
    out_type=[jax.ShapeDtypeStruct((NPAD, CW), jnp.float32)] * NCHUNK,
    mesh=_sc_mesh,
    scratch_types=[
        pltpu.VMEM((K, 128), jnp.int32),
        pltpu.VMEM((K, 128), jnp.int32),
        pltpu.VMEM((K, 128), jnp.int32),
        pltpu.VMEM((K, 128), jnp.int32),
        pltpu.VMEM((K * 128, CW), jnp.float32),
        pltpu.VMEM((K * 128, CW), jnp.float32),
        pltpu.VMEM((ZCH, CW), jnp.float32),
        pltpu.VMEM_SHARED((NPAD, CW), jnp.float32),
        pltpu.SemaphoreType.DMA,
        pltpu.SemaphoreType.DMA,
    ],
    compiler_params=pltpu.CompilerParams(use_tc_tiling_on_sc=False),
    name="gin_agg",
)


def _pool_body(h, bat_g, bat_t, out, hbuf, bidx, tidx, zp, pacc, psem):
    del psem
    cid = lax.axis_index("c")
    sid = lax.axis_index("s")
    w = sid * NC + cid

    for i in range(16):
        for c in range(4):
            zp[i, pl.ds(c * 16, 16)] = jnp.zeros((16,), jnp.float32)
    pltpu.sync_copy(zp, pacc.at[pl.ds(sid * 32, 16)])
    pltpu.sync_copy(zp, pacc.at[pl.ds(sid * 32 + 16, 16)])
    plsc.subcore_barrier()

    pltpu.sync_copy(bat_g.at[pl.ds(w * GPW, GPW)], bidx)

    def gb(i, _):
        g = w * GPW + i
        pltpu.sync_copy(h.at[pl.ds(g * 128, 128)], hbuf)
        pltpu.sync_copy(hbuf, pacc.at[bidx.at[i]], add=True)
        return 0
    lax.fori_loop(0, GPW, gb, 0)

    @pl.when(w == 0)
    def _rem():
        pltpu.sync_copy(bat_g.at[pl.ds(NW * GPW, NGR)],
                        bidx.at[pl.ds(0, NGR)])

        def rb(i, _):
            g = NW * GPW + i
            pltpu.sync_copy(h.at[pl.ds(g * 128, 128)], hbuf)
            pltpu.sync_copy(hbuf, pacc.at[bidx.at[i]], add=True)
            return 0
        lax.fori_loop(0, NGR, rb, 0)

    @pl.when(w == NW - 1)
    def _tail():
        pltpu.sync_copy(h.at[pl.ds(NGF * 128, TAIL)], hbuf.at[pl.ds(0, TAIL)])
        pltpu.sync_copy(bat_t, tidx)
        pltpu.sync_copy(hbuf.at[pl.ds(0, TAIL)], pacc.at[tidx.at[0]],
                        add=True)
    plsc.subcore_barrier()

    pltpu.sync_copy(pacc.at[pl.ds(sid * 32, 32)], hbuf.at[pl.ds(0, 32)])
    for cc in range(NC):
        @pl.when(cid == cc)
        def _flush(cc=cc):
            pltpu.sync_copy(hbuf.at[pl.ds(0, 32)],
                            out.at[cc].at[pl.ds(sid * 32, 32)])


_pool = pl.kernel(
    _pool_body,
    out_type=jax.ShapeDtypeStruct((NC, G, H), jnp.float32),
    mesh=_sc_mesh,
    scratch_types=[
        pltpu.VMEM((128, H), jnp.float32),
        pltpu.VMEM((GPW, 128), jnp.int32),
        pltpu.VMEM((1, TAIL), jnp.int32),
        pltpu.VMEM((16, H), jnp.float32),
        pltpu.VMEM_SHARED((G, H), jnp.float32),
        pltpu.SemaphoreType.DMA,
    ],
    compiler_params=pltpu.CompilerParams(use_tc_tiling_on_sc=False),
    name="gin_pool",
)


BN = 800


def _lin1_body(x_ref, w_ref, o0, o1, o2, o3):
    z = jnp.dot(x_ref[...], w_ref[...], preferred_element_type=jnp.float32)
    for c, o in enumerate((o0, o1, o2, o3)):
        o[...] = z[:, c * CW:(c + 1) * CW]


_lin1 = pl.pallas_call(
    _lin1_body,
    grid=(N // BN,),
    in_specs=[pl.BlockSpec((BN, F_IN), lambda i: (i, 0)),
              pl.BlockSpec((F_IN, H), lambda i: (0, 0))],
    out_specs=[pl.BlockSpec((BN, CW), lambda i: (i, 0))] * NCHUNK,
    out_shape=[jax.ShapeDtypeStruct((N, CW), jnp.float32)] * NCHUNK,
)


def _ep_core(z, a, b1r, scr, ber, w2r, b2r):
    pre = (z + a + b1r[...]) * scr[...] + ber[...]
    t = jnp.maximum(pre, 0.0)
    u = jnp.dot(t, w2r[...], preferred_element_type=jnp.float32) + b2r[...]
    return jnp.maximum(u, 0.0)


def _ep_mid_body(z0, z1, z2, z3, a0, a1, a2, a3,
                 b1r, scr, ber, w2r, b2r, w1nr,
                 hr, n0, n1, n2, n3):
    z = jnp.concatenate([z0[...], z1[...], z2[...], z3[...]], axis=1)
    a = jnp.concatenate([a0[...], a1[...], a2[...], a3[...]], axis=1)
    u = _ep_core(z, a, b1r, scr, ber, w2r, b2r)
    hr[...] = u
    zn = jnp.dot(u, w1nr[...], preferred_element_type=jnp.float32)
    for c, o in enumerate((n0, n1, n2, n3)):
        o[...] = zn[:, c * CW:(c + 1) * CW]


def _ep_last_body(z0, z1, z2, z3, a0, a1, a2, a3,
                  b1r, scr, ber, w2r, b2r, hr):
    z = jnp.concatenate([z0[...], z1[...], z2[...], z3[...]], axis=1)
    a = jnp.concatenate([a0[...], a1[...], a2[...], a3[...]], axis=1)
    hr[...] = _ep_core(z, a, b1r, scr, ber, w2r, b2r)


_chunk_spec = pl.BlockSpec((BN, CW), lambda i: (i, 0))
_vec_spec = pl.BlockSpec((1, H), lambda i: (0, 0))
_mat_spec = pl.BlockSpec((H, H), lambda i: (0, 0))

_ep_mid = pl.pallas_call(
    _ep_mid_body,
    grid=(N // BN,),
    in_specs=[_chunk_spec] * 8 + [_vec_spec] * 3 + [_mat_spec, _vec_spec,
                                                    _mat_spec],
    out_specs=[pl.BlockSpec((BN, H), lambda i: (i, 0))] + [_chunk_spec] * 4,
    out_shape=[jax.ShapeDtypeStruct((N, H), jnp.float32)]
    + [jax.ShapeDtypeStruct((N, CW), jnp.float32)] * NCHUNK,
)

_ep_last = pl.pallas_call(
    _ep_last_body,
    grid=(N // BN,),
    in_specs=[_chunk_spec] * 8 + [_vec_spec] * 3 + [_mat_spec, _vec_spec],
    out_specs=pl.BlockSpec((BN, H), lambda i: (i, 0)),
    out_shape=jax.ShapeDtypeStruct((N, H), jnp.float32),
)


def _fin_body(p1r, p2r, p3r, w1r, b1r, w2r, b2r, w3r, b3r, outr):
    ps = [r[...][0] + r[...][1] for r in (p1r, p2r, p3r)]
    hcat = jnp.concatenate(ps, axis=1)
    t = jnp.dot(hcat, w1r[...], preferred_element_type=jnp.float32) + b1r[...]
    t = jnp.maximum(t, 0.0)
    t = jnp.dot(t, w2r[...], preferred_element_type=jnp.float32) + b2r[...]
    t = jnp.maximum(t, 0.0)
    t = jnp.dot(t, w3r[...], preferred_element_type=jnp.float32) + b3r[...]
    outr[...] = 1.0 / (1.0 + jnp.exp(-t))


def _full(shape):
    return pl.BlockSpec(shape, lambda: tuple(0 for _ in shape))


_fin = pl.pallas_call(
    _fin_body,
    in_specs=[_full((NC, G, H))] * 3
    + [_full((3 * H, 50)), _full((1, 50)), _full((50, 20)), _full((1, 20)),
       _full((20, 1)), _full((1, 1))],
    out_specs=_full((G, 1)),
    out_shape=jax.ShapeDtypeStruct((G, 1), jnp.float32),
)


def kernel(x, edge_index, batch,
           W1_1, b1_1, gamma_1, beta_1, W2_1, b2_1,
           W1_2, b1_2, gamma_2, beta_2, W2_2, b2_2,
           W1_3, b1_3, gamma_3, beta_3, W2_3, b2_3,
           Wl1, bl1, Wl2, bl2, Wl3, bl3):
    src = edge_index[0]
    dst = edge_index[1]
    pad = EP - E
    src_g = jnp.concatenate([src, jnp.zeros((pad,), jnp.int32)]).reshape(
        EG, 128)
    dst_g = jnp.concatenate([dst, jnp.full((pad,), N, jnp.int32)]).reshape(
        EG, 128)
    bat_g = batch[:NGF * 128].reshape(NGF, 128)
    bat_t = batch[NGF * 128:].reshape(1, TAIL)

    inv = 1.0 / jnp.sqrt(jnp.float32(1.0 + 1e-5))
    r = lambda v: v.reshape(1, -1)

    z1 = _lin1(x, W1_1)
    a1 = _agg(*z1, src_g, dst_g)
    h1, *z2 = _ep_mid(*z1, *a1, r(b1_1), r(gamma_1 * inv), r(beta_1),
                      W2_1, r(b2_1), W1_2)
    a2 = _agg(*z2, src_g, dst_g)
    h2, *z3 = _ep_mid(*z2, *a2, r(b1_2), r(gamma_2 * inv), r(beta_2),
                      W2_2, r(b2_2), W1_3)
    a3 = _agg(*z3, src_g, dst_g)
    h3 = _ep_last(*z3, *a3, r(b1_3), r(gamma_3 * inv), r(beta_3),
                  W2_3, r(b2_3))
    p1 = _pool(h1, bat_g, bat_t)
    p2 = _pool(h2, bat_g, bat_t)
    p3 = _pool(h3, bat_g, bat_t)
    return _fin(p1, p2, p3, Wl1, r(bl1), Wl2, r(bl2), Wl3, r(bl3))


# speedup vs baseline: 1.2150x; 1.0314x over previous
import functools

import jax
import jax.numpy as jnp
from jax import lax
from jax.experimental import pallas as pl
from jax.experimental.pallas import tpu as pltpu
from jax.experimental.pallas import tpu_sc as plsc

N = 100000
E = 1600000
F_IN = 33
H = 64
G = 512

NC, NS = 2, 16
NW = NC * NS
CW = 16
NCHUNK = H // CW
EP = 1638400
EG = EP // 128
GPS = EG // NS
K = 4
NBLK = GPS // K
NPAIR = NBLK // 2
NPAD = 100096
RPS = NPAD // NS
ZCH = 184

NGF = N // 128
GPW = 24
NGR = NGF - NW * GPW
TAIL = N - NGF * 128

_sc_mesh = plsc.VectorSubcoreMesh(core_axis_name="c", subcore_axis_name="s")


def _agg_body(z0, z1, z2, z3, src_g, dst_g,
              o0, o1, o2, o3,
              srcv0, dstv0, srcv1, dstv1, rows0, rows1, zv, acc,
              gsem, ssem):
    cid = lax.axis_index("c")
    sid = lax.axis_index("s")
    zs = (z0, z1, z2, z3)
    outs = (o0, o1, o2, o3)

    def _zfill(i, _):
        zv[i, :] = jnp.zeros((CW,), jnp.float32)
        return 0
    lax.fori_loop(0, ZCH, _zfill, 0)

    for jj in range(NCHUNK):
        @pl.when(cid == (jj % NC))
        def _chunk(jj=jj):
            zt = zs[jj]
            ot = outs[jj]
            for t in range(RPS // ZCH):
                pltpu.sync_copy(zv, acc.at[pl.ds(sid * RPS + t * ZCH, ZCH)])
            plsc.subcore_barrier()

            def load_and_fire(b, sv, dv, rv):
                g0 = sid * GPS + b * K
                pltpu.sync_copy(src_g.at[pl.ds(g0, K)], sv)
                pltpu.sync_copy(dst_g.at[pl.ds(g0, K)], dv)
                for k in range(K):
                    pltpu.async_copy(zt.at[sv.at[k]],
                                     rv.at[pl.ds(k * 128, 128)], gsem)

            def drain_gathers(sv, rv):
                for k in range(K):
                    pltpu.make_async_copy(
                        zt.at[sv.at[k]],
                        rv.at[pl.ds(k * 128, 128)], gsem).wait()

            def fire_scatters(dv, rv):
                for k in range(K):
                    pltpu.async_copy(rv.at[pl.ds(k * 128, 128)],
                                     acc.at[dv.at[k]], ssem, add=True)

            def drain_scatters(dv, rv):
                for k in range(K):
                    pltpu.make_async_copy(rv.at[pl.ds(k * 128, 128)],
                                          acc.at[dv.at[k]], ssem).wait()

            load_and_fire(0, srcv0, dstv0, rows0)

            def pip(i, _):
                @pl.when(i > 0)
                def _d1():
                    drain_scatters(dstv1, rows1)
                load_and_fire(2 * i + 1, srcv1, dstv1, rows1)
                drain_gathers(srcv0, rows0)
                fire_scatters(dstv0, rows0)

                @pl.when(i < NPAIR - 1)
                def _n0():
                    drain_scatters(dstv0, rows0)
                    load_and_fire(2 * i + 2, srcv0, dstv0, rows0)
                drain_gathers(srcv1, rows1)
                fire_scatters(dstv1, rows1)
                return 0
            lax.fori_loop(0, NPAIR, pip, 0)
            drain_scatters(dstv0, rows0)
            drain_scatters(dstv1, rows1)
            plsc.subcore_barrier()

            for t in range(RPS // ZCH):
                r0 = sid * RPS + t * ZCH
                pltpu.sync_copy(acc.at[pl.ds(r0, ZCH)],
                                rows0.at[pl.ds(0, ZCH)])
                pltpu.sync_copy(rows0.at[pl.ds(0, ZCH)],
                                ot.at[pl.ds(r0, ZCH)])
            plsc.subcore_barrier()


_agg = pl.kernel(
    _agg_body,
    out_type=[jax.ShapeDtypeStruct((NPAD, CW), jnp.float32)] * NCHUNK,
    mesh=_sc_mesh,
    scratch_types=[
        pltpu.VMEM((K, 128), jnp.int32),
        pltpu.VMEM((K, 128), jnp.int32),
        pltpu.VMEM((K, 128), jnp.int32),
        pltpu.VMEM((K, 128), jnp.int32),
        pltpu.VMEM((K * 128, CW), jnp.float32),
        pltpu.VMEM((K * 128, CW), jnp.float32),
        pltpu.VMEM((ZCH, CW), jnp.float32),
        pltpu.VMEM_SHARED((NPAD, CW), jnp.float32),
        pltpu.SemaphoreType.DMA,
        pltpu.SemaphoreType.DMA,
    ],
    compiler_params=pltpu.CompilerParams(use_tc_tiling_on_sc=False),
    name="gin_agg",
)


def _pool_body(h, bat_g, bat_t, out, hbuf, bidx, tidx, zp, pacc, psem):
    del psem
    cid = lax.axis_index("c")
    sid = lax.axis_index("s")
    w = sid * NC + cid

    for i in range(16):
        for c in range(4):
            zp[i, pl.ds(c * 16, 16)] = jnp.zeros((16,), jnp.float32)
    pltpu.sync_copy(zp, pacc.at[pl.ds(sid * 32, 16)])
    pltpu.sync_copy(zp, pacc.at[pl.ds(sid * 32 + 16, 16)])
    plsc.subcore_barrier()

    pltpu.sync_copy(bat_g.at[pl.ds(w * GPW, GPW)], bidx)

    def gb(i, _):
        g = w * GPW + i
        pltpu.sync_copy(h.at[pl.ds(g * 128, 128)], hbuf)
        pltpu.sync_copy(hbuf, pacc.at[bidx.at[i]], add=True)
        return 0
    lax.fori_loop(0, GPW, gb, 0)

    @pl.when(w == 0)
    def _rem():
        pltpu.sync_copy(bat_g.at[pl.ds(NW * GPW, NGR)],
                        bidx.at[pl.ds(0, NGR)])

        def rb(i, _):
            g = NW * GPW + i
            pltpu.sync_copy(h.at[pl.ds(g * 128, 128)], hbuf)
            pltpu.sync_copy(hbuf, pacc.at[bidx.at[i]], add=True)
            return 0
        lax.fori_loop(0, NGR, rb, 0)

    @pl.when(w == NW - 1)
    def _tail():
        pltpu.sync_copy(h.at[pl.ds(NGF * 128, TAIL)], hbuf.at[pl.ds(0, TAIL)])
        pltpu.sync_copy(bat_t, tidx)
        pltpu.sync_copy(hbuf.at[pl.ds(0, TAIL)], pacc.at[tidx.at[0]],
                        add=True)
    plsc.subcore_barrier()

    pltpu.sync_copy(pacc.at[pl.ds(sid * 32, 32)], hbuf.at[pl.ds(0, 32)])
    for cc in range(NC):
        @pl.when(cid == cc)
        def _flush(cc=cc):
            pltpu.sync_copy(hbuf.at[pl.ds(0, 32)],
                            out.at[cc].at[pl.ds(sid * 32, 32)])


_pool = pl.kernel(
    _pool_body,
    out_type=jax.ShapeDtypeStruct((NC, G, H), jnp.float32),
    mesh=_sc_mesh,
    scratch_types=[
        pltpu.VMEM((128, H), jnp.float32),
        pltpu.VMEM((GPW, 128), jnp.int32),
        pltpu.VMEM((1, TAIL), jnp.int32),
        pltpu.VMEM((16, H), jnp.float32),
        pltpu.VMEM_SHARED((G, H), jnp.float32),
        pltpu.SemaphoreType.DMA,
    ],
    compiler_params=pltpu.CompilerParams(use_tc_tiling_on_sc=False),
    name="gin_pool",
)


BN = 2000


def _lin1_body(x_ref, w_ref, o0, o1, o2, o3):
    z = jnp.dot(x_ref[...], w_ref[...], preferred_element_type=jnp.float32)
    for c, o in enumerate((o0, o1, o2, o3)):
        o[...] = z[:, c * CW:(c + 1) * CW]


_lin1 = pl.pallas_call(
    _lin1_body,
    grid=(N // BN,),
    in_specs=[pl.BlockSpec((BN, F_IN), lambda i: (i, 0)),
              pl.BlockSpec((F_IN, H), lambda i: (0, 0))],
    out_specs=[pl.BlockSpec((BN, CW), lambda i: (i, 0))] * NCHUNK,
    out_shape=[jax.ShapeDtypeStruct((N, CW), jnp.float32)] * NCHUNK,
)


def _ep_core(z, a, b1r, scr, ber, w2r, b2r):
    pre = (z + a + b1r[...]) * scr[...] + ber[...]
    t = jnp.maximum(pre, 0.0)
    u = jnp.dot(t, w2r[...], preferred_element_type=jnp.float32) + b2r[...]
    return jnp.maximum(u, 0.0)


def _ep_mid_body(z0, z1, z2, z3, a0, a1, a2, a3,
                 b1r, scr, ber, w2r, b2r, w1nr,
                 hr, n0, n1, n2, n3):
    z = jnp.concatenate([z0[...], z1[...], z2[...], z3[...]], axis=1)
    a = jnp.concatenate([a0[...], a1[...], a2[...], a3[...]], axis=1)
    u = _ep_core(z, a, b1r, scr, ber, w2r, b2r)
    hr[...] = u
    zn = jnp.dot(u, w1nr[...], preferred_element_type=jnp.float32)
    for c, o in enumerate((n0, n1, n2, n3)):
        o[...] = zn[:, c * CW:(c + 1) * CW]


def _ep_last_body(z0, z1, z2, z3, a0, a1, a2, a3,
                  b1r, scr, ber, w2r, b2r, hr):
    z = jnp.concatenate([z0[...], z1[...], z2[...], z3[...]], axis=1)
    a = jnp.concatenate([a0[...], a1[...], a2[...], a3[...]], axis=1)
    hr[...] = _ep_core(z, a, b1r, scr, ber, w2r, b2r)


_chunk_spec = pl.BlockSpec((BN, CW), lambda i: (i, 0))
_vec_spec = pl.BlockSpec((1, H), lambda i: (0, 0))
_mat_spec = pl.BlockSpec((H, H), lambda i: (0, 0))

_ep_mid = pl.pallas_call(
    _ep_mid_body,
    grid=(N // BN,),
    in_specs=[_chunk_spec] * 8 + [_vec_spec] * 3 + [_mat_spec, _vec_spec,
                                                    _mat_spec],
    out_specs=[pl.BlockSpec((BN, H), lambda i: (i, 0))] + [_chunk_spec] * 4,
    out_shape=[jax.ShapeDtypeStruct((N, H), jnp.float32)]
    + [jax.ShapeDtypeStruct((N, CW), jnp.float32)] * NCHUNK,
)

_ep_last = pl.pallas_call(
    _ep_last_body,
    grid=(N // BN,),
    in_specs=[_chunk_spec] * 8 + [_vec_spec] * 3 + [_mat_spec, _vec_spec],
    out_specs=pl.BlockSpec((BN, H), lambda i: (i, 0)),
    out_shape=jax.ShapeDtypeStruct((N, H), jnp.float32),
)


def _fin_body(p1r, p2r, p3r, w1r, b1r, w2r, b2r, w3r, b3r, outr):
    ps = [r[...][0] + r[...][1] for r in (p1r, p2r, p3r)]
    hcat = jnp.concatenate(ps, axis=1)
    t = jnp.dot(hcat, w1r[...], preferred_element_type=jnp.float32) + b1r[...]
    t = jnp.maximum(t, 0.0)
    t = jnp.dot(t, w2r[...], preferred_element_type=jnp.float32) + b2r[...]
    t = jnp.maximum(t, 0.0)
    t = jnp.dot(t, w3r[...], preferred_element_type=jnp.float32) + b3r[...]
    outr[...] = 1.0 / (1.0 + jnp.exp(-t))


def _full(shape):
    return pl.BlockSpec(shape, lambda: tuple(0 for _ in shape))


_fin = pl.pallas_call(
    _fin_body,
    in_specs=[_full((NC, G, H))] * 3
    + [_full((3 * H, 50)), _full((1, 50)), _full((50, 20)), _full((1, 20)),
       _full((20, 1)), _full((1, 1))],
    out_specs=_full((G, 1)),
    out_shape=jax.ShapeDtypeStruct((G, 1), jnp.float32),
)


def kernel(x, edge_index, batch,
           W1_1, b1_1, gamma_1, beta_1, W2_1, b2_1,
           W1_2, b1_2, gamma_2, beta_2, W2_2, b2_2,
           W1_3, b1_3, gamma_3, beta_3, W2_3, b2_3,
           Wl1, bl1, Wl2, bl2, Wl3, bl3):
    src = edge_index[0]
    dst = edge_index[1]
    pad = EP - E
    src_g = jnp.concatenate([src, jnp.zeros((pad,), jnp.int32)]).reshape(
        EG, 128)
    dst_g = jnp.concatenate([dst, jnp.full((pad,), N, jnp.int32)]).reshape(
        EG, 128)
    bat_g = batch[:NGF * 128].reshape(NGF, 128)
    bat_t = batch[NGF * 128:].reshape(1, TAIL)

    inv = 1.0 / jnp.sqrt(jnp.float32(1.0 + 1e-5))
    r = lambda v: v.reshape(1, -1)

    z1 = _lin1(x, W1_1)
    a1 = _agg(*z1, src_g, dst_g)
    h1, *z2 = _ep_mid(*z1, *a1, r(b1_1), r(gamma_1 * inv), r(beta_1),
                      W2_1, r(b2_1), W1_2)
    a2 = _agg(*z2, src_g, dst_g)
    h2, *z3 = _ep_mid(*z2, *a2, r(b1_2), r(gamma_2 * inv), r(beta_2),
                      W2_2, r(b2_2), W1_3)
    a3 = _agg(*z3, src_g, dst_g)
    h3 = _ep_last(*z3, *a3, r(b1_3), r(gamma_3 * inv), r(beta_3),
                  W2_3, r(b2_3))
    p1 = _pool(h1, bat_g, bat_t)
    p2 = _pool(h2, bat_g, bat_t)
    p3 = _pool(h3, bat_g, bat_t)
    return _fin(p1, p2, p3, Wl1, r(bl1), Wl2, r(bl2), Wl3, r(bl3))


# speedup vs baseline: 1.6316x; 1.3429x over previous
import functools

import jax
import jax.numpy as jnp
from jax import lax
from jax.experimental import pallas as pl
from jax.experimental.pallas import tpu as pltpu
from jax.experimental.pallas import tpu_sc as plsc

N = 100000
E = 1600000
F_IN = 33
H = 64
G = 512

NC, NS = 2, 16
NW = NC * NS
CW = 16
NCHUNK = H // CW
EP = 1638400
EG = EP // 128
GPS = EG // NS
K = 4
NBLK = GPS // K
NPAIR = NBLK // 2
NPAD = 100096
RPS = NPAD // NS
ZCH = 184

NGF = N // 128
GPW = 24
NGR = NGF - NW * GPW
TAIL = N - NGF * 128

_sc_mesh = plsc.VectorSubcoreMesh(core_axis_name="c", subcore_axis_name="s")


def _agg_body(z0, z1, z2, z3, src_g, dst_g,
              o0, o1, o2, o3,
              srcv0, dstv0, srcv1, dstv1, rows0, rows1, zv, acc,
              gsem, ssem):
    cid = lax.axis_index("c")
    sid = lax.axis_index("s")
    zs = (z0, z1, z2, z3)
    outs = (o0, o1, o2, o3)

    def _zfill(i, _):
        zv[i, :] = jnp.zeros((CW,), jnp.float32)
        return 0
    lax.fori_loop(0, ZCH, _zfill, 0)

    for jj in range(NCHUNK):
        @pl.when(cid == (jj % NC))
        def _chunk(jj=jj):
            zt = zs[jj]
            ot = outs[jj]
            for t in range(RPS // ZCH):
                pltpu.sync_copy(zv, acc.at[pl.ds(sid * RPS + t * ZCH, ZCH)])
            plsc.subcore_barrier()

            def load_and_fire(b, sv, dv, rv):
                g0 = sid * GPS + b * K
                pltpu.sync_copy(src_g.at[pl.ds(g0, K)], sv)
                pltpu.sync_copy(dst_g.at[pl.ds(g0, K)], dv)
                for k in range(K):
                    pltpu.async_copy(zt.at[sv.at[k]],
                                     rv.at[pl.ds(k * 128, 128)], gsem)

            def drain_gathers(sv, rv):
                for k in range(K):
                    pltpu.make_async_copy(
                        zt.at[sv.at[k]],
                        rv.at[pl.ds(k * 128, 128)], gsem).wait()

            def fire_scatters(dv, rv):
                for k in range(K):
                    pltpu.async_copy(rv.at[pl.ds(k * 128, 128)],
                                     acc.at[dv.at[k]], ssem, add=True)

            def drain_scatters(dv, rv):
                for k in range(K):
                    pltpu.make_async_copy(rv.at[pl.ds(k * 128, 128)],
                                          acc.at[dv.at[k]], ssem).wait()

            load_and_fire(0, srcv0, dstv0, rows0)

            def pip(i, _):
                @pl.when(i > 0)
                def _d1():
                    drain_scatters(dstv1, rows1)
                load_and_fire(2 * i + 1, srcv1, dstv1, rows1)
                drain_gathers(srcv0, rows0)
                fire_scatters(dstv0, rows0)

                @pl.when(i < NPAIR - 1)
                def _n0():
                    drain_scatters(dstv0, rows0)
                    load_and_fire(2 * i + 2, srcv0, dstv0, rows0)
                drain_gathers(srcv1, rows1)
                fire_scatters(dstv1, rows1)
                return 0
            lax.fori_loop(0, NPAIR, pip, 0)
            drain_scatters(dstv0, rows0)
            drain_scatters(dstv1, rows1)
            plsc.subcore_barrier()

            for t in range(RPS // ZCH):
                r0 = sid * RPS + t * ZCH
                pltpu.sync_copy(acc.at[pl.ds(r0, ZCH)],
                                rows0.at[pl.ds(0, ZCH)])
                pltpu.sync_copy(rows0.at[pl.ds(0, ZCH)],
                                ot.at[pl.ds(r0, ZCH)])
            plsc.subcore_barrier()


_agg = pl.kernel(
    _agg_body,
    out_type=[jax.ShapeDtypeStruct((NPAD, CW), jnp.float32)] * NCHUNK,
    mesh=_sc_mesh,
    scratch_types=[
        pltpu.VMEM((K, 128), jnp.int32),
        pltpu.VMEM((K, 128), jnp.int32),
        pltpu.VMEM((K, 128), jnp.int32),
        pltpu.VMEM((K, 128), jnp.int32),
        pltpu.VMEM((K * 128, CW), jnp.float32),
        pltpu.VMEM((K * 128, CW), jnp.float32),
        pltpu.VMEM((ZCH, CW), jnp.float32),
        pltpu.VMEM_SHARED((NPAD, CW), jnp.float32),
        pltpu.SemaphoreType.DMA,
        pltpu.SemaphoreType.DMA,
    ],
    compiler_params=pltpu.CompilerParams(use_tc_tiling_on_sc=False),
    name="gin_agg",
)


def _pool_body(h0, h1, h2, h3, bat_g, bat_t, out, hbuf, bidx, tidx, zp,
               pa0, pa1, pa2, pa3, psem):
    del psem
    cid = lax.axis_index("c")
    sid = lax.axis_index("s")
    w = sid * NC + cid
    hs = (h0, h1, h2, h3)
    pas = (pa0, pa1, pa2, pa3)

    for i in range(32):
        zp[i, :] = jnp.zeros((CW,), jnp.float32)
    for pa in pas:
        pltpu.sync_copy(zp, pa.at[pl.ds(sid * 32, 32)])
    plsc.subcore_barrier()

    pltpu.sync_copy(bat_g.at[pl.ds(w * GPW, GPW)], bidx)

    def gb(i, _):
        g = w * GPW + i
        for hc, pa in zip(hs, pas):
            pltpu.sync_copy(hc.at[pl.ds(g * 128, 128)], hbuf)
            pltpu.sync_copy(hbuf, pa.at[bidx.at[i]], add=True)
        return 0
    lax.fori_loop(0, GPW, gb, 0)

    @pl.when(w == 0)
    def _rem():
        pltpu.sync_copy(bat_g.at[pl.ds(NW * GPW, NGR)],
                        bidx.at[pl.ds(0, NGR)])

        def rb(i, _):
            g = NW * GPW + i
            for hc, pa in zip(hs, pas):
                pltpu.sync_copy(hc.at[pl.ds(g * 128, 128)], hbuf)
                pltpu.sync_copy(hbuf, pa.at[bidx.at[i]], add=True)
            return 0
        lax.fori_loop(0, NGR, rb, 0)

    @pl.when(w == NW - 1)
    def _tail():
        pltpu.sync_copy(bat_t, tidx)
        for hc, pa in zip(hs, pas):
            pltpu.sync_copy(hc.at[pl.ds(NGF * 128, TAIL)],
                            hbuf.at[pl.ds(0, TAIL)])
            pltpu.sync_copy(hbuf.at[pl.ds(0, TAIL)], pa.at[tidx.at[0]],
                            add=True)
    plsc.subcore_barrier()

    for cc in range(NC):
        @pl.when(cid == cc)
        def _flush(cc=cc):
            for c, pa in enumerate(pas):
                pltpu.sync_copy(pa.at[pl.ds(sid * 32, 32)],
                                hbuf.at[pl.ds(0, 32)])
                pltpu.sync_copy(hbuf.at[pl.ds(0, 32)],
                                out.at[cc * NCHUNK + c].at[pl.ds(sid * 32,
                                                                 32)])


_pool = pl.kernel(
    _pool_body,
    out_type=jax.ShapeDtypeStruct((NC * NCHUNK, G, CW), jnp.float32),
    mesh=_sc_mesh,
    scratch_types=[
        pltpu.VMEM((128, CW), jnp.float32),
        pltpu.VMEM((GPW, 128), jnp.int32),
        pltpu.VMEM((1, TAIL), jnp.int32),
        pltpu.VMEM((32, CW), jnp.float32),
        pltpu.VMEM_SHARED((G, CW), jnp.float32),
        pltpu.VMEM_SHARED((G, CW), jnp.float32),
        pltpu.VMEM_SHARED((G, CW), jnp.float32),
        pltpu.VMEM_SHARED((G, CW), jnp.float32),
        pltpu.SemaphoreType.DMA,
    ],
    compiler_params=pltpu.CompilerParams(use_tc_tiling_on_sc=False),
    name="gin_pool",
)


BN = 2944


_pk_spec = pl.BlockSpec((BN // 8, 128), lambda i: (i, 0))


def _lin1_body(x_ref, w_ref, o0, o1, o2, o3):
    z = jnp.dot(x_ref[...], w_ref[...], preferred_element_type=jnp.float32)
    for c, o in enumerate((o0, o1, o2, o3)):
        o[...] = z[:, c * 128:(c + 1) * 128]


_lin1 = pl.pallas_call(
    _lin1_body,
    grid=(NPAD // BN,),
    in_specs=[pl.BlockSpec((BN // 8, 8 * F_IN), lambda i: (i, 0)),
              pl.BlockSpec((8 * F_IN, 512), lambda i: (0, 0))],
    out_specs=[_pk_spec] * NCHUNK,
    out_shape=[jax.ShapeDtypeStruct((NPAD // 8, 128), jnp.float32)] * NCHUNK,
)


def _ep_core(z0, z1, z2, z3, a0, a1, a2, a3, b1r, scr, ber, m2r, b2r):
    z = jnp.concatenate([z0[...], z1[...], z2[...], z3[...]], axis=1)
    a = jnp.concatenate([a0[...], a1[...], a2[...], a3[...]], axis=1)
    pre = (z + a + b1r[...]) * scr[...] + ber[...]
    t = jnp.maximum(pre, 0.0)
    u = jnp.dot(t, m2r[...], preferred_element_type=jnp.float32) + b2r[...]
    return jnp.maximum(u, 0.0)


def _ep_mid_body(z0, z1, z2, z3, a0, a1, a2, a3,
                 b1r, scr, ber, m2r, b2r, m1nr,
                 h0, h1, h2, h3, n0, n1, n2, n3):
    u = _ep_core(z0, z1, z2, z3, a0, a1, a2, a3, b1r, scr, ber, m2r, b2r)
    for c, o in enumerate((h0, h1, h2, h3)):
        o[...] = u[:, c * 128:(c + 1) * 128]
    zn = jnp.dot(u, m1nr[...], preferred_element_type=jnp.float32)
    for c, o in enumerate((n0, n1, n2, n3)):
        o[...] = zn[:, c * 128:(c + 1) * 128]


def _ep_last_body(z0, z1, z2, z3, a0, a1, a2, a3,
                  b1r, scr, ber, m2r, b2r, h0, h1, h2, h3):
    u = _ep_core(z0, z1, z2, z3, a0, a1, a2, a3, b1r, scr, ber, m2r, b2r)
    for c, o in enumerate((h0, h1, h2, h3)):
        o[...] = u[:, c * 128:(c + 1) * 128]


_vec_spec = pl.BlockSpec((1, 512), lambda i: (0, 0))
_mat_spec = pl.BlockSpec((512, 512), lambda i: (0, 0))

_ep_mid = pl.pallas_call(
    _ep_mid_body,
    grid=(NPAD // BN,),
    in_specs=[_pk_spec] * 8 + [_vec_spec] * 3 + [_mat_spec, _vec_spec,
                                                 _mat_spec],
    out_specs=[_pk_spec] * 8,
    out_shape=[jax.ShapeDtypeStruct((NPAD // 8, 128), jnp.float32)] * 8,
)

_ep_last = pl.pallas_call(
    _ep_last_body,
    grid=(NPAD // BN,),
    in_specs=[_pk_spec] * 8 + [_vec_spec] * 3 + [_mat_spec, _vec_spec],
    out_specs=[_pk_spec] * 4,
    out_shape=[jax.ShapeDtypeStruct((NPAD // 8, 128), jnp.float32)] * 4,
)


def _fin_body(p1r, p2r, p3r, w1r, b1r, w2r, b2r, w3r, b3r, outr):
    ps = []
    for r in (p1r, p2r, p3r):
        q = r[...]
        ps.append(jnp.concatenate(
            [q[c] + q[NCHUNK + c] for c in range(NCHUNK)], axis=1))
    hcat = jnp.concatenate(ps, axis=1)
    t = jnp.dot(hcat, w1r[...], preferred_element_type=jnp.float32) + b1r[...]
    t = jnp.maximum(t, 0.0)
    t = jnp.dot(t, w2r[...], preferred_element_type=jnp.float32) + b2r[...]
    t = jnp.maximum(t, 0.0)
    t = jnp.dot(t, w3r[...], preferred_element_type=jnp.float32) + b3r[...]
    outr[...] = 1.0 / (1.0 + jnp.exp(-t))


def _full(shape):
    return pl.BlockSpec(shape, lambda: tuple(0 for _ in shape))


_fin = pl.pallas_call(
    _fin_body,
    in_specs=[_full((NC * NCHUNK, G, CW))] * 3
    + [_full((3 * H, 50)), _full((1, 50)), _full((50, 20)), _full((1, 20)),
       _full((20, 1)), _full((1, 1))],
    out_specs=_full((G, 1)),
    out_shape=jax.ShapeDtypeStruct((G, 1), jnp.float32),
)


def kernel(x, edge_index, batch,
           W1_1, b1_1, gamma_1, beta_1, W2_1, b2_1,
           W1_2, b1_2, gamma_2, beta_2, W2_2, b2_2,
           W1_3, b1_3, gamma_3, beta_3, W2_3, b2_3,
           Wl1, bl1, Wl2, bl2, Wl3, bl3):
    src = edge_index[0]
    dst = edge_index[1]
    pad = EP - E
    src_g = jnp.concatenate([src, jnp.zeros((pad,), jnp.int32)]).reshape(
        EG, 128)
    dst_g = jnp.concatenate([dst, jnp.full((pad,), N, jnp.int32)]).reshape(
        EG, 128)
    bat_g = batch[:NGF * 128].reshape(NGF, 128)
    bat_t = batch[NGF * 128:].reshape(1, TAIL)

    inv = 1.0 / jnp.sqrt(jnp.float32(1.0 + 1e-5))
    r = lambda v: v.reshape(1, -1)
    eye8 = jnp.eye(8, dtype=jnp.float32)

    def bd1(W):
        return jnp.einsum('fcg,jk->jfckg', W.reshape(F_IN, NCHUNK, CW),
                          eye8).reshape(8 * F_IN, 512)

    def bd(W):
        return jnp.einsum('dgcm,jk->djgckm',
                          W.reshape(NCHUNK, CW, NCHUNK, CW),
                          eye8).reshape(512, 512)

    def vp(v):
        return jnp.tile(v.reshape(NCHUNK, 1, CW), (1, 8, 1)).reshape(1, 512)

    sc_view = lambda zs: tuple(jnp.reshape(z, (NPAD, CW)) for z in zs)
    tc_view = lambda as_: tuple(jnp.reshape(a, (NPAD // 8, 128)) for a in as_)

    xp = jnp.concatenate([x, jnp.zeros((NPAD - N, F_IN), jnp.float32)])
    z1 = _lin1(xp.reshape(NPAD // 8, 8 * F_IN), bd1(W1_1))
    a1 = tc_view(_agg(*sc_view(z1), src_g, dst_g))
    o1 = _ep_mid(*z1, *a1, vp(b1_1), vp(gamma_1 * inv), vp(beta_1),
                 bd(W2_1), vp(b2_1), bd(W1_2))
    h1, z2 = o1[:NCHUNK], o1[NCHUNK:]
    a2 = tc_view(_agg(*sc_view(z2), src_g, dst_g))
    o2 = _ep_mid(*z2, *a2, vp(b1_2), vp(gamma_2 * inv), vp(beta_2),
                 bd(W2_2), vp(b2_2), bd(W1_3))
    h2, z3 = o2[:NCHUNK], o2[NCHUNK:]
    a3 = tc_view(_agg(*sc_view(z3), src_g, dst_g))
    h3 = _ep_last(*z3, *a3, vp(b1_3), vp(gamma_3 * inv), vp(beta_3),
                  bd(W2_3), vp(b2_3))
    p1 = _pool(*sc_view(h1), bat_g, bat_t)
    p2 = _pool(*sc_view(h2), bat_g, bat_t)
    p3 = _pool(*sc_view(h3), bat_g, bat_t)
    return _fin(p1, p2, p3, Wl1, r(bl1), Wl2, r(bl2), Wl3, r(bl3))
